# Initial kernel scaffold; baseline (speedup 1.0000x reference)
import functools

import jax
import jax.numpy as jnp
from jax import lax
from jax.experimental import pallas as pl
from jax.experimental.pallas import tpu as pltpu
from jax.experimental.pallas import tpu_sc as plsc

K = 5
V = 512
C = 16
NG = 16
NEB = 8192
NODES = NG * V
NK = K * K
EPT = NEB // 16
GPS = NG // 2


def _prep_body(xg_ref, wcat_ref, b2_ref, bres2_ref, ea0_ref, ea1_ref,
               src_ref, eexp_ref, y_ref, root_ref, res_ref, basisb_ref,
               idx4_ref):
    xg = xg_ref[...]
    p = lax.dot_general(xg, wcat_ref[...], (((1,), (0,)), ((), ())),
                        preferred_element_type=jnp.float32)
    y_ref[...] = p[:, :NK * C]
    root_ref[...] = p[:, NK * C:NK * C + C] + b2_ref[...]
    r = p[:, NK * C + C:] + bres2_ref[...]
    res_ref[...] = jnp.where(r > 0, r, jnp.expm1(r))

    pos0 = ea0_ref[...] * (K - 1.0)
    pos1 = ea1_ref[...] * (K - 1.0)
    i0f = jnp.clip(jnp.floor(pos0), 0.0, K - 2.0)
    i1f = jnp.clip(jnp.floor(pos1), 0.0, K - 2.0)
    f0 = pos0 - i0f
    f1 = pos1 - i1f
    g0 = 1.0 - f0
    g1 = 1.0 - f1
    basis4 = jnp.concatenate([g0 * g1, f0 * g1, g0 * f1, f0 * f1], axis=1)
    basisb_ref[...] = jnp.dot(basis4, eexp_ref[...],
                              preferred_element_type=jnp.float32)
    base = (src_ref[...] * NK + i0f.astype(jnp.int32)
            + i1f.astype(jnp.int32) * K)
    idx4_ref[...] = jnp.concatenate(
        [base, base + 1, base + K, base + K + 1], axis=1)


_prep = pl.pallas_call(
    _prep_body,
    out_shape=(
        jax.ShapeDtypeStruct((NODES, NK * C), jnp.float32),
        jax.ShapeDtypeStruct((NODES, C), jnp.float32),
        jax.ShapeDtypeStruct((NODES, C), jnp.float32),
        jax.ShapeDtypeStruct((NEB, 4 * 16), jnp.float32),
        jax.ShapeDtypeStruct((NEB, 4), jnp.int32),
    ),
)


def _cnt_body(dst_ref, cnt_ref):
    dst = dst_ref[...]
    iot = lax.broadcasted_iota(jnp.int32, (V, 1), 0)
    oneh = (iot == dst).astype(jnp.float32)
    cnt_ref[...] = jnp.dot(oneh, jnp.ones((NEB, 1), jnp.float32),
                           preferred_element_type=jnp.float32)


_cnt = pl.pallas_call(
    _cnt_body,
    out_shape=jax.ShapeDtypeStruct((V, 1), jnp.float32),
)


def _final_body(agg_ref, cnt_ref, root_ref, res_ref, o_ref):
    cnt = jnp.maximum(cnt_ref[...], 1.0)
    agg = agg_ref[...].reshape(NG, V, C)
    root = root_ref[...].reshape(NG, V, C)
    res = res_ref[...].reshape(NG, V, C)
    xo = agg / cnt[None] + root
    xo = jnp.where(xo > 0, xo, jnp.expm1(xo))
    xo = xo + res
    xo = jnp.where(xo > 0, xo, jnp.expm1(xo))
    o_ref[...] = xo.reshape(NODES, C)


_final = pl.pallas_call(
    _final_body,
    out_shape=jax.ShapeDtypeStruct((NODES, C), jnp.float32),
)


_mesh = plsc.VectorSubcoreMesh(core_axis_name="c", subcore_axis_name="s")


@functools.partial(
    pl.kernel,
    mesh=_mesh,
    out_type=jax.ShapeDtypeStruct((NODES, C), jnp.float32),
    scratch_types=[
        pltpu.VMEM((4 * EPT,), jnp.int32),
        pltpu.VMEM((EPT,), jnp.int32),
        pltpu.VMEM((EPT * 64,), jnp.float32),
        pltpu.VMEM((EPT, C), jnp.float32),
        pltpu.VMEM((EPT, C), jnp.float32),
        pltpu.VMEM((EPT, C), jnp.float32),
        pltpu.VMEM((EPT, C), jnp.float32),
        pltpu.VMEM((EPT, C), jnp.float32),
        pltpu.VMEM((GPS * V // 16, C), jnp.float32),
        pltpu.VMEM_SHARED((GPS * V, C), jnp.float32),
        pltpu.SemaphoreType.DMA,
    ],
)
def _edges(y_hbm, idxf_hbm, basisf_hbm, dst_hbm, agg_hbm,
           idx_v, dst_v, basis_v, r0, r1, r2, r3, m_v, tmp_v, agg_sp, sem):
    c = lax.axis_index("c")
    s = lax.axis_index("s")
    e0 = s * EPT
    rows_per_tile = GPS * V // 16

    def _zb(j, carry):
        tmp_v[j] = jnp.zeros((C,), jnp.float32)
        return carry
    lax.fori_loop(0, rows_per_tile, _zb, 0)
    pltpu.sync_copy(tmp_v, agg_sp.at[pl.ds(s * rows_per_tile, rows_per_tile)])

    for corner in range(4):
        pltpu.sync_copy(idxf_hbm.at[pl.ds(corner * NEB + e0, EPT)],
                        idx_v.at[pl.ds(corner * EPT, EPT)])
    pltpu.sync_copy(dst_hbm.at[pl.ds(e0, EPT)], dst_v)
    pltpu.sync_copy(basisf_hbm.at[pl.ds(e0 * 64, EPT * 64)], basis_v)

    def _addv(ref, nchunks, val):
        def f(j, carry):
            ref[pl.ds(j * 16, 16)] = ref[pl.ds(j * 16, 16)] + val
            return carry
        lax.fori_loop(0, nchunks, f, 0)

    _addv(idx_v, 4 * EPT // 16, c * (GPS * V * NK))
    plsc.subcore_barrier()

    for g in range(GPS):
        descs = [
            pltpu.async_copy(y_hbm.at[idx_v.at[pl.ds(k * EPT, EPT)]], r, sem)
            for k, r in enumerate((r0, r1, r2, r3))
        ]
        for d in descs:
            d.wait()

        def _body(e, carry):
            m = (r0[e] * basis_v[pl.ds(e * 64, 16)]
                 + r1[e] * basis_v[pl.ds(e * 64 + 16, 16)]
                 + r2[e] * basis_v[pl.ds(e * 64 + 32, 16)]
                 + r3[e] * basis_v[pl.ds(e * 64 + 48, 16)])
            m_v[e] = m
            return carry
        lax.fori_loop(0, EPT, _body, 0)

        pltpu.sync_copy(m_v, agg_sp.at[dst_v], add=True)
        if g < GPS - 1:
            _addv(idx_v, 4 * EPT // 16, V * NK)
            _addv(dst_v, EPT // 16, V)

    plsc.subcore_barrier()
    pltpu.sync_copy(agg_sp.at[pl.ds(s * rows_per_tile, rows_per_tile)], tmp_v)
    pltpu.sync_copy(
        tmp_v,
        agg_hbm.at[pl.ds(c * GPS * V + s * rows_per_tile, rows_per_tile)])


def kernel(x, edge_index, edge_attr, Wspline, Wroot, b, Wres, bres):
    n, v, cc, t = x.shape
    xg = x.transpose(3, 0, 1, 2).reshape(NODES, C)

    wflat = Wspline.transpose(1, 0, 2).reshape(C, NK * C)
    wcat = jnp.concatenate([wflat, Wroot, Wres.T], axis=1)
    b2 = b.reshape(1, C)
    bres2 = bres.reshape(1, C)

    ea = edge_attr[:NEB]
    ea0 = ea[:, 0:1]
    ea1 = ea[:, 1:2]
    src = edge_index[0, :NEB].reshape(NEB, 1)
    dst = edge_index[1, :NEB]

    eexp = jnp.repeat(jnp.eye(4, dtype=jnp.float32), 16, axis=1)

    y, root, res, basisb, idx4 = _prep(xg, wcat, b2, bres2, ea0, ea1,
                                       src, eexp)
    yflat = y.reshape(NODES * NK, C)
    idxf = idx4.T.reshape(-1)
    basisf = basisb.reshape(-1)

    cnt = _cnt(dst.reshape(1, NEB))
    agg = _edges(yflat, idxf, basisf, dst)
    out_node = _final(agg, cnt, root, res)

    return out_node.reshape(t, n, v, C).transpose(1, 2, 3, 0)


# speedup vs baseline: 48.0883x; 48.0883x over previous
import functools

import jax
import jax.numpy as jnp
from jax import lax
from jax.experimental import pallas as pl
from jax.experimental.pallas import tpu as pltpu
from jax.experimental.pallas import tpu_sc as plsc

K = 5
V = 512
C = 16
NG = 16
NEB = 8192
NODES = NG * V
NK = K * K
EPT = NEB // 16
GPS = NG // 2


def _prep_body(xg_ref, wcat_ref, b2_ref, bres2_ref, ea0_ref, ea1_ref,
               src_ref, eexp_ref, y_ref, root_ref, res_ref, basisb_ref,
               idx4_ref):
    xg = xg_ref[...]
    p = lax.dot_general(xg, wcat_ref[...], (((1,), (0,)), ((), ())),
                        preferred_element_type=jnp.float32)
    y_ref[...] = p[:, :NK * C]
    root_ref[...] = p[:, NK * C:NK * C + C] + b2_ref[...]
    r = p[:, NK * C + C:] + bres2_ref[...]
    res_ref[...] = jnp.where(r > 0, r, jnp.exp(r) - 1.0)

    pos0 = ea0_ref[...] * (K - 1.0)
    pos1 = ea1_ref[...] * (K - 1.0)
    i0f = jnp.clip(jnp.floor(pos0), 0.0, K - 2.0)
    i1f = jnp.clip(jnp.floor(pos1), 0.0, K - 2.0)
    f0 = pos0 - i0f
    f1 = pos1 - i1f
    g0 = 1.0 - f0
    g1 = 1.0 - f1
    basis4 = jnp.concatenate([g0 * g1, f0 * g1, g0 * f1, f0 * f1], axis=1)
    basisb_ref[...] = jnp.dot(basis4, eexp_ref[...],
                              preferred_element_type=jnp.float32)
    base = (src_ref[...] * NK + i0f.astype(jnp.int32)
            + i1f.astype(jnp.int32) * K)
    idx4_ref[...] = jnp.concatenate(
        [base, base + 1, base + K, base + K + 1], axis=1)


_PREP_BLK = 1024
_prep = pl.pallas_call(
    _prep_body,
    grid=(NODES // _PREP_BLK,),
    in_specs=[
        pl.BlockSpec((_PREP_BLK, C), lambda i: (i, 0)),
        pl.BlockSpec((C, 432), lambda i: (0, 0)),
        pl.BlockSpec((1, C), lambda i: (0, 0)),
        pl.BlockSpec((1, C), lambda i: (0, 0)),
        pl.BlockSpec((_PREP_BLK, 1), lambda i: (i, 0)),
        pl.BlockSpec((_PREP_BLK, 1), lambda i: (i, 0)),
        pl.BlockSpec((_PREP_BLK, 1), lambda i: (i, 0)),
        pl.BlockSpec((4, 64), lambda i: (0, 0)),
    ],
    out_specs=[
        pl.BlockSpec((_PREP_BLK, NK * C), lambda i: (i, 0)),
        pl.BlockSpec((_PREP_BLK, C), lambda i: (i, 0)),
        pl.BlockSpec((_PREP_BLK, C), lambda i: (i, 0)),
        pl.BlockSpec((_PREP_BLK, 4 * 16), lambda i: (i, 0)),
        pl.BlockSpec((_PREP_BLK, 4), lambda i: (i, 0)),
    ],
    out_shape=(
        jax.ShapeDtypeStruct((NODES, NK * C), jnp.float32),
        jax.ShapeDtypeStruct((NODES, C), jnp.float32),
        jax.ShapeDtypeStruct((NODES, C), jnp.float32),
        jax.ShapeDtypeStruct((NEB, 4 * 16), jnp.float32),
        jax.ShapeDtypeStruct((NEB, 4), jnp.int32),
    ),
)


def _cnt_body(dst_ref, cnt_ref):
    dst = dst_ref[...]
    iot = lax.broadcasted_iota(jnp.int32, (V, 1), 0)
    oneh = (iot == dst).astype(jnp.float32)
    cnt_ref[...] = jnp.dot(oneh, jnp.ones((NEB, 1), jnp.float32),
                           preferred_element_type=jnp.float32)


_cnt = pl.pallas_call(
    _cnt_body,
    out_shape=jax.ShapeDtypeStruct((V, 1), jnp.float32),
)


def _final_body(agg_ref, cnt_ref, root_ref, res_ref, o_ref):
    cnt = jnp.maximum(cnt_ref[...], 1.0)
    agg = agg_ref[...].reshape(NG, V, C)
    root = root_ref[...].reshape(NG, V, C)
    res = res_ref[...].reshape(NG, V, C)
    xo = agg / cnt[None] + root
    xo = jnp.where(xo > 0, xo, jnp.exp(xo) - 1.0)
    xo = xo + res
    xo = jnp.where(xo > 0, xo, jnp.exp(xo) - 1.0)
    o_ref[...] = xo.reshape(NODES, C)


_final = pl.pallas_call(
    _final_body,
    out_shape=jax.ShapeDtypeStruct((NODES, C), jnp.float32),
)


def _edges_body(y_hbm, idxf_hbm, basisf_hbm, dst_hbm, agg_hbm,
                i0_v, i1_v, i2_v, i3_v, dst_v, basis_v, r0, r1, r2, r3,
                m_v, tmp_v, agg_sp, sem):
    c = lax.axis_index("c")
    s = lax.axis_index("s")
    e0 = s * EPT
    rows_per_tile = GPS * V // 16

    def _zb(j, carry):
        tmp_v[j] = jnp.zeros((C,), jnp.float32)
        return carry
    lax.fori_loop(0, rows_per_tile, _zb, 0)
    pltpu.sync_copy(tmp_v, agg_sp.at[pl.ds(s * rows_per_tile, rows_per_tile)])

    idx_refs = (i0_v, i1_v, i2_v, i3_v)
    for corner in range(4):
        pltpu.sync_copy(idxf_hbm.at[pl.ds(corner * NEB + e0, EPT)],
                        idx_refs[corner])
    pltpu.sync_copy(dst_hbm.at[pl.ds(e0, EPT)], dst_v)
    pltpu.sync_copy(basisf_hbm.at[pl.ds(e0 * 64, EPT * 64)], basis_v)

    def _addv(ref, nchunks, val):
        def f(j, carry):
            ref[pl.ds(j * 16, 16)] = ref[pl.ds(j * 16, 16)] + val
            return carry
        lax.fori_loop(0, nchunks, f, 0)

    for corner in range(4):
        _addv(idx_refs[corner], EPT // 16, c * (GPS * V * NK))
    plsc.subcore_barrier()

    for g in range(GPS):
        descs = [
            pltpu.async_copy(y_hbm.at[ix], r, sem)
            for ix, r in zip(idx_refs, (r0, r1, r2, r3))
        ]
        for d in descs:
            d.wait()

        def _body(e, carry):
            m = (r0[e] * basis_v[pl.ds(e * 64, 16)]
                 + r1[e] * basis_v[pl.ds(e * 64 + 16, 16)]
                 + r2[e] * basis_v[pl.ds(e * 64 + 32, 16)]
                 + r3[e] * basis_v[pl.ds(e * 64 + 48, 16)])
            m_v[e] = m
            return carry
        lax.fori_loop(0, EPT, _body, 0)

        pltpu.sync_copy(m_v, agg_sp.at[dst_v], add=True)
        if g < GPS - 1:
            for corner in range(4):
                _addv(idx_refs[corner], EPT // 16, V * NK)
            _addv(dst_v, EPT // 16, V)

    plsc.subcore_barrier()
    pltpu.sync_copy(agg_sp.at[pl.ds(s * rows_per_tile, rows_per_tile)], tmp_v)
    pltpu.sync_copy(
        tmp_v,
        agg_hbm.at[pl.ds(c * GPS * V + s * rows_per_tile, rows_per_tile)])


@functools.cache
def _get_edges():
    mesh = plsc.VectorSubcoreMesh(core_axis_name="c", subcore_axis_name="s",
                                  num_cores=2, num_subcores=16)
    return pl.kernel(
        _edges_body,
        mesh=mesh,
        compiler_params=pltpu.CompilerParams(use_tc_tiling_on_sc=False),
        out_type=jax.ShapeDtypeStruct((NODES, C), jnp.float32),
        scratch_types=[
            pltpu.VMEM((EPT,), jnp.int32),
            pltpu.VMEM((EPT,), jnp.int32),
            pltpu.VMEM((EPT,), jnp.int32),
            pltpu.VMEM((EPT,), jnp.int32),
            pltpu.VMEM((EPT,), jnp.int32),
            pltpu.VMEM((EPT * 64,), jnp.float32),
            pltpu.VMEM((EPT, C), jnp.float32),
            pltpu.VMEM((EPT, C), jnp.float32),
            pltpu.VMEM((EPT, C), jnp.float32),
            pltpu.VMEM((EPT, C), jnp.float32),
            pltpu.VMEM((EPT, C), jnp.float32),
            pltpu.VMEM((GPS * V // 16, C), jnp.float32),
            pltpu.VMEM_SHARED((GPS * V, C), jnp.float32),
            pltpu.SemaphoreType.DMA,
        ],
    )


def kernel(x, edge_index, edge_attr, Wspline, Wroot, b, Wres, bres):
    n, v, cc, t = x.shape
    xg = x.transpose(3, 0, 1, 2).reshape(NODES, C)

    wflat = Wspline.transpose(1, 0, 2).reshape(C, NK * C)
    wcat = jnp.concatenate([wflat, Wroot, Wres.T], axis=1)
    b2 = b.reshape(1, C)
    bres2 = bres.reshape(1, C)

    ea = edge_attr[:NEB]
    ea0 = ea[:, 0:1]
    ea1 = ea[:, 1:2]
    src = edge_index[0, :NEB].reshape(NEB, 1)
    dst = edge_index[1, :NEB]

    eexp = jnp.repeat(jnp.eye(4, dtype=jnp.float32), 16, axis=1)

    y, root, res, basisb, idx4 = _prep(xg, wcat, b2, bres2, ea0, ea1,
                                       src, eexp)
    yflat = y.reshape(NODES * NK, C)
    idxf = idx4.T.reshape(-1)
    basisf = basisb.reshape(-1)

    cnt = _cnt(dst.reshape(1, NEB))
    agg = _get_edges()(yflat, idxf, basisf, dst)
    out_node = _final(agg, cnt, root, res)

    return out_node.reshape(t, n, v, C).transpose(1, 2, 3, 0)


# speedup vs baseline: 54.9356x; 1.1424x over previous
import functools

import jax
import jax.numpy as jnp
from jax import lax
from jax.experimental import pallas as pl
from jax.experimental.pallas import tpu as pltpu
from jax.experimental.pallas import tpu_sc as plsc

K = 5
V = 512
C = 16
NG = 16
NEB = 8192
NODES = NG * V
NK = K * K
EPT = NEB // 16
GPS = NG // 2


def _prep_body(xg_ref, wcat_ref, b2_ref, bres2_ref, ea0_ref, ea1_ref,
               src_ref, eexp_ref, y_ref, root_ref, res_ref, basisb_ref,
               idx4_ref):
    xg = xg_ref[...]
    p = lax.dot_general(xg, wcat_ref[...], (((1,), (0,)), ((), ())),
                        preferred_element_type=jnp.float32)
    y_ref[...] = p[:, :NK * C]
    root_ref[...] = p[:, NK * C:NK * C + C] + b2_ref[...]
    r = p[:, NK * C + C:] + bres2_ref[...]
    res_ref[...] = jnp.where(r > 0, r, jnp.exp(r) - 1.0)

    pos0 = ea0_ref[...] * (K - 1.0)
    pos1 = ea1_ref[...] * (K - 1.0)
    i0f = jnp.clip(jnp.floor(pos0), 0.0, K - 2.0)
    i1f = jnp.clip(jnp.floor(pos1), 0.0, K - 2.0)
    f0 = pos0 - i0f
    f1 = pos1 - i1f
    g0 = 1.0 - f0
    g1 = 1.0 - f1
    basis4 = jnp.concatenate([g0 * g1, f0 * g1, g0 * f1, f0 * f1], axis=1)
    basisb_ref[...] = jnp.dot(basis4, eexp_ref[...],
                              preferred_element_type=jnp.float32)
    base = (src_ref[...] * NK + i0f.astype(jnp.int32)
            + i1f.astype(jnp.int32) * K)
    idx4_ref[...] = jnp.concatenate(
        [base, base + 1, base + K, base + K + 1], axis=1)


_PREP_BLK = 1024
_prep = pl.pallas_call(
    _prep_body,
    grid=(NODES // _PREP_BLK,),
    in_specs=[
        pl.BlockSpec((_PREP_BLK, C), lambda i: (i, 0)),
        pl.BlockSpec((C, 432), lambda i: (0, 0)),
        pl.BlockSpec((1, C), lambda i: (0, 0)),
        pl.BlockSpec((1, C), lambda i: (0, 0)),
        pl.BlockSpec((_PREP_BLK, 1), lambda i: (i, 0)),
        pl.BlockSpec((_PREP_BLK, 1), lambda i: (i, 0)),
        pl.BlockSpec((_PREP_BLK, 1), lambda i: (i, 0)),
        pl.BlockSpec((4, 64), lambda i: (0, 0)),
    ],
    out_specs=[
        pl.BlockSpec((_PREP_BLK, NK * C), lambda i: (i, 0)),
        pl.BlockSpec((_PREP_BLK, C), lambda i: (i, 0)),
        pl.BlockSpec((_PREP_BLK, C), lambda i: (i, 0)),
        pl.BlockSpec((_PREP_BLK, 4 * 16), lambda i: (i, 0)),
        pl.BlockSpec((_PREP_BLK, 4), lambda i: (i, 0)),
    ],
    out_shape=(
        jax.ShapeDtypeStruct((NODES, NK * C), jnp.float32),
        jax.ShapeDtypeStruct((NODES, C), jnp.float32),
        jax.ShapeDtypeStruct((NODES, C), jnp.float32),
        jax.ShapeDtypeStruct((NEB, 4 * 16), jnp.float32),
        jax.ShapeDtypeStruct((NEB, 4), jnp.int32),
    ),
)


def _edges_body(y_hbm, idxf_hbm, basisf_hbm, dst_hbm, root_hbm, res_hbm,
                out_hbm,
                ia0, ia1, ia2, ia3, ib0, ib1, ib2, ib3, dst_v, basis_v,
                ra0, ra1, ra2, ra3, rb0, rb1, rb2, rb3,
                m_v, agg_sp, cnt_sp, sem_a, sem_b):
    c = lax.axis_index("c")
    s = lax.axis_index("s")
    e0 = s * EPT
    rpt = GPS * V // 16

    idx_a = (ia0, ia1, ia2, ia3)
    idx_b = (ib0, ib1, ib2, ib3)
    rows_a = (ra0, ra1, ra2, ra3)
    rows_b = (rb0, rb1, rb2, rb3)

    def _fill(j, carry):
        m_v[j] = jnp.zeros((C,), jnp.float32)
        ra0[j] = jnp.ones((C,), jnp.float32)
        ra0[j + rpt] = jnp.ones((C,), jnp.float32)
        return carry
    lax.fori_loop(0, rpt, _fill, 0)

    pltpu.sync_copy(m_v.at[pl.ds(0, rpt)], agg_sp.at[pl.ds(s * rpt, rpt)])
    pltpu.sync_copy(m_v.at[pl.ds(0, V // 16)],
                    cnt_sp.at[pl.ds(s * (V // 16), V // 16)])

    for corner in range(4):
        pltpu.sync_copy(idxf_hbm.at[pl.ds(corner * NEB + e0, EPT)],
                        idx_a[corner])
    pltpu.sync_copy(dst_hbm.at[pl.ds(e0, EPT)], dst_v)
    pltpu.sync_copy(basisf_hbm.at[pl.ds(e0 * 64, EPT * 64)], basis_v)

    def _addv(dref, sref, nchunks, val):
        def f(j, carry):
            dref[pl.ds(j * 16, 16)] = sref[pl.ds(j * 16, 16)] + val
            return carry
        lax.fori_loop(0, nchunks, f, 0)

    for corner in range(4):
        _addv(idx_a[corner], idx_a[corner], EPT // 16, c * (GPS * V * NK))
    plsc.subcore_barrier()

    pltpu.sync_copy(ra0, cnt_sp.at[dst_v], add=True)

    bufs = ((idx_a, rows_a, sem_a), (idx_b, rows_b, sem_b))
    descs = [pltpu.async_copy(y_hbm.at[ix], r, sem_a)
             for ix, r in zip(idx_a, rows_a)]
    for g in range(GPS):
        cur_i, cur_r, _ = bufs[g % 2]
        nxt_i, nxt_r, nxt_s = bufs[(g + 1) % 2]
        if g < GPS - 1:
            for corner in range(4):
                _addv(nxt_i[corner], cur_i[corner], EPT // 16, V * NK)
        for d in descs:
            d.wait()
        if g < GPS - 1:
            descs = [pltpu.async_copy(y_hbm.at[ix], r, nxt_s)
                     for ix, r in zip(nxt_i, nxt_r)]

        c0, c1, c2, c3 = cur_r

        def _body(e, carry):
            for u in range(4):
                ee = e * 4 + u
                m = (c0[ee] * basis_v[pl.ds(ee * 64, 16)]
                     + c1[ee] * basis_v[pl.ds(ee * 64 + 16, 16)]
                     + c2[ee] * basis_v[pl.ds(ee * 64 + 32, 16)]
                     + c3[ee] * basis_v[pl.ds(ee * 64 + 48, 16)])
                m_v[ee] = m
            return carry
        lax.fori_loop(0, EPT // 4, _body, 0)

        pltpu.sync_copy(m_v, agg_sp.at[dst_v], add=True)
        if g < GPS - 1:
            _addv(dst_v, dst_v, EPT // 16, V)

    plsc.subcore_barrier()

    row0 = s * rpt
    gbase = c * (GPS * V) + row0
    v_off = lax.rem(s, 2) * rpt
    pltpu.sync_copy(agg_sp.at[pl.ds(row0, rpt)], m_v.at[pl.ds(0, rpt)])
    pltpu.sync_copy(cnt_sp.at[pl.ds(v_off, rpt)], ra2.at[pl.ds(0, rpt)])
    pltpu.sync_copy(root_hbm.at[pl.ds(gbase, rpt)], ra0.at[pl.ds(0, rpt)])
    pltpu.sync_copy(res_hbm.at[pl.ds(gbase, rpt)], ra1.at[pl.ds(0, rpt)])

    def _fin(j, carry):
        cntv = jnp.maximum(ra2[j], 1.0)
        xo = m_v[j] / cntv + ra0[j]
        xo = jnp.where(xo > 0, xo, jnp.exp(xo) - 1.0)
        xo = xo + ra1[j]
        xo = jnp.where(xo > 0, xo, jnp.exp(xo) - 1.0)
        m_v[j + rpt] = xo
        return carry
    lax.fori_loop(0, rpt, _fin, 0)
    pltpu.sync_copy(m_v.at[pl.ds(rpt, rpt)], out_hbm.at[pl.ds(gbase, rpt)])


@functools.cache
def _get_edges():
    mesh = plsc.VectorSubcoreMesh(core_axis_name="c", subcore_axis_name="s",
                                  num_cores=2, num_subcores=16)
    idx_t = pltpu.VMEM((EPT,), jnp.int32)
    row_t = pltpu.VMEM((EPT, C), jnp.float32)
    return pl.kernel(
        _edges_body,
        mesh=mesh,
        compiler_params=pltpu.CompilerParams(use_tc_tiling_on_sc=False),
        out_type=jax.ShapeDtypeStruct((NODES, C), jnp.float32),
        scratch_types=[
            idx_t, idx_t, idx_t, idx_t,
            idx_t, idx_t, idx_t, idx_t,
            idx_t,
            pltpu.VMEM((EPT * 64,), jnp.float32),
            row_t, row_t, row_t, row_t,
            row_t, row_t, row_t, row_t,
            row_t,
            pltpu.VMEM_SHARED((GPS * V, C), jnp.float32),
            pltpu.VMEM_SHARED((V, C), jnp.float32),
            pltpu.SemaphoreType.DMA,
            pltpu.SemaphoreType.DMA,
        ],
    )


def kernel(x, edge_index, edge_attr, Wspline, Wroot, b, Wres, bres):
    n, v, cc, t = x.shape
    xg = x.transpose(3, 0, 1, 2).reshape(NODES, C)

    wflat = Wspline.transpose(1, 0, 2).reshape(C, NK * C)
    wcat = jnp.concatenate([wflat, Wroot, Wres.T], axis=1)
    b2 = b.reshape(1, C)
    bres2 = bres.reshape(1, C)

    ea = edge_attr[:NEB]
    ea0 = ea[:, 0:1]
    ea1 = ea[:, 1:2]
    src = edge_index[0, :NEB].reshape(NEB, 1)
    dst = edge_index[1, :NEB]

    eexp = jnp.repeat(jnp.eye(4, dtype=jnp.float32), 16, axis=1)

    y, root, res, basisb, idx4 = _prep(xg, wcat, b2, bres2, ea0, ea1,
                                       src, eexp)
    yflat = y.reshape(NODES * NK, C)
    idxf = idx4.T.reshape(-1)
    basisf = basisb.reshape(-1)

    out_node = _get_edges()(yflat, idxf, basisf, dst, root, res)

    return out_node.reshape(t, n, v, C).transpose(1, 2, 3, 0)


# speedup vs baseline: 55.0081x; 1.0013x over previous
import functools

import jax
import jax.numpy as jnp
from jax import lax
from jax.experimental import pallas as pl
from jax.experimental.pallas import tpu as pltpu
from jax.experimental.pallas import tpu_sc as plsc

K = 5
V = 512
C = 16
NG = 16
NEB = 8192
NODES = NG * V
NK = K * K
EPT = NEB // 16
GPS = NG // 2


def _prep_body(xg_ref, wcat_ref, b2_ref, bres2_ref, ea0_ref, ea1_ref,
               src_ref, eexp_ref, y_ref, root_ref, res_ref, basisb_ref,
               idx4_ref):
    xg = xg_ref[...]
    p = lax.dot_general(xg, wcat_ref[...], (((1,), (0,)), ((), ())),
                        preferred_element_type=jnp.float32)
    y_ref[...] = p[:, :NK * C]
    root_ref[...] = p[:, NK * C:NK * C + C] + b2_ref[...]
    r = p[:, NK * C + C:] + bres2_ref[...]
    res_ref[...] = jnp.where(r > 0, r, jnp.exp(r) - 1.0)

    pos0 = ea0_ref[...] * (K - 1.0)
    pos1 = ea1_ref[...] * (K - 1.0)
    i0f = jnp.clip(jnp.floor(pos0), 0.0, K - 2.0)
    i1f = jnp.clip(jnp.floor(pos1), 0.0, K - 2.0)
    f0 = pos0 - i0f
    f1 = pos1 - i1f
    g0 = 1.0 - f0
    g1 = 1.0 - f1
    basis4 = jnp.concatenate([g0 * g1, f0 * g1, g0 * f1, f0 * f1], axis=1)
    basisb_ref[...] = jnp.dot(basis4, eexp_ref[...],
                              preferred_element_type=jnp.float32)
    base = (src_ref[...] * NK + i0f.astype(jnp.int32)
            + i1f.astype(jnp.int32) * K)
    idx4_ref[...] = jnp.concatenate(
        [base, base + 1, base + K, base + K + 1], axis=1)


_PREP_BLK = 1024
_prep = pl.pallas_call(
    _prep_body,
    grid=(NODES // _PREP_BLK,),
    in_specs=[
        pl.BlockSpec((_PREP_BLK, C), lambda i: (i, 0)),
        pl.BlockSpec((C, 432), lambda i: (0, 0)),
        pl.BlockSpec((1, C), lambda i: (0, 0)),
        pl.BlockSpec((1, C), lambda i: (0, 0)),
        pl.BlockSpec((_PREP_BLK, 1), lambda i: (i, 0)),
        pl.BlockSpec((_PREP_BLK, 1), lambda i: (i, 0)),
        pl.BlockSpec((_PREP_BLK, 1), lambda i: (i, 0)),
        pl.BlockSpec((4, 64), lambda i: (0, 0)),
    ],
    out_specs=[
        pl.BlockSpec((_PREP_BLK, NK * C), lambda i: (i, 0)),
        pl.BlockSpec((_PREP_BLK, C), lambda i: (i, 0)),
        pl.BlockSpec((_PREP_BLK, C), lambda i: (i, 0)),
        pl.BlockSpec((_PREP_BLK, 4 * 16), lambda i: (i, 0)),
        pl.BlockSpec((_PREP_BLK, 4), lambda i: (i, 0)),
    ],
    out_shape=(
        jax.ShapeDtypeStruct((NODES, NK * C), jnp.float32),
        jax.ShapeDtypeStruct((NODES, C), jnp.float32),
        jax.ShapeDtypeStruct((NODES, C), jnp.float32),
        jax.ShapeDtypeStruct((NEB, 4 * 16), jnp.float32),
        jax.ShapeDtypeStruct((NEB, 4), jnp.int32),
    ),
)


def _edges_body(y_hbm, idxf_hbm, basisf_hbm, dst_hbm, root_hbm, res_hbm,
                out_hbm,
                ia0, ia1, ia2, ia3, ib0, ib1, ib2, ib3, dst_v, basis_v,
                ra0, ra1, ra2, ra3, rb0, rb1, rb2, rb3,
                m_v, agg_sp, cnt_sp, sem_a, sem_b):
    c = lax.axis_index("c")
    s = lax.axis_index("s")
    e0 = s * EPT
    rpt = GPS * V // 16

    idx_a = (ia0, ia1, ia2, ia3)
    idx_b = (ib0, ib1, ib2, ib3)
    rows_a = (ra0, ra1, ra2, ra3)
    rows_b = (rb0, rb1, rb2, rb3)

    def _fill(j, carry):
        m_v[j] = jnp.zeros((C,), jnp.float32)
        ra0[j] = jnp.ones((C,), jnp.float32)
        ra0[j + rpt] = jnp.ones((C,), jnp.float32)
        return carry
    lax.fori_loop(0, rpt, _fill, 0)

    pltpu.sync_copy(m_v.at[pl.ds(0, rpt)], agg_sp.at[pl.ds(s * rpt, rpt)])
    pltpu.sync_copy(m_v.at[pl.ds(0, V // 16)],
                    cnt_sp.at[pl.ds(s * (V // 16), V // 16)])

    for corner in range(4):
        pltpu.sync_copy(idxf_hbm.at[pl.ds(corner * NEB + e0, EPT)],
                        idx_a[corner])
    pltpu.sync_copy(dst_hbm.at[pl.ds(e0, EPT)], dst_v)
    pltpu.sync_copy(basisf_hbm.at[pl.ds(e0 * 64, EPT * 64)], basis_v)

    def _addv(dref, sref, nchunks, val):
        def f(j, carry):
            dref[pl.ds(j * 16, 16)] = sref[pl.ds(j * 16, 16)] + val
            return carry
        lax.fori_loop(0, nchunks, f, 0)

    for corner in range(4):
        _addv(idx_a[corner], idx_a[corner], EPT // 16, c * (GPS * V * NK))
    plsc.subcore_barrier()

    pltpu.sync_copy(ra0, cnt_sp.at[dst_v], add=True)

    bufs = ((idx_a, rows_a, sem_a), (idx_b, rows_b, sem_b))
    descs = [pltpu.async_copy(y_hbm.at[ix], r, sem_a)
             for ix, r in zip(idx_a, rows_a)]
    for g in range(GPS):
        cur_i, cur_r, _ = bufs[g % 2]
        nxt_i, nxt_r, nxt_s = bufs[(g + 1) % 2]
        if g < GPS - 1:
            with jax.named_scope("idxprep"):
                for corner in range(4):
                    _addv(nxt_i[corner], cur_i[corner], EPT // 16, V * NK)
        with jax.named_scope("dwait"):
            for d in descs:
                d.wait()
        if g < GPS - 1:
            descs = [pltpu.async_copy(y_hbm.at[ix], r, nxt_s)
                     for ix, r in zip(nxt_i, nxt_r)]

        c0, c1, c2, c3 = cur_r

        def _body(e, carry):
            for u in range(4):
                ee = e * 4 + u
                m = (c0[ee] * basis_v[pl.ds(ee * 64, 16)]
                     + c1[ee] * basis_v[pl.ds(ee * 64 + 16, 16)]
                     + c2[ee] * basis_v[pl.ds(ee * 64 + 32, 16)]
                     + c3[ee] * basis_v[pl.ds(ee * 64 + 48, 16)])
                m_v[ee] = m
            return carry
        with jax.named_scope("fma"):
            lax.fori_loop(0, EPT // 4, _body, 0)

        with jax.named_scope("scat"):
            pltpu.sync_copy(m_v, agg_sp.at[dst_v], add=True)
        if g < GPS - 1:
            with jax.named_scope("dstinc"):
                _addv(dst_v, dst_v, EPT // 16, V)

    plsc.subcore_barrier()

    row0 = s * rpt
    gbase = c * (GPS * V) + row0
    v_off = lax.rem(s, 2) * rpt
    pltpu.sync_copy(agg_sp.at[pl.ds(row0, rpt)], m_v.at[pl.ds(0, rpt)])
    pltpu.sync_copy(cnt_sp.at[pl.ds(v_off, rpt)], ra2.at[pl.ds(0, rpt)])
    pltpu.sync_copy(root_hbm.at[pl.ds(gbase, rpt)], ra0.at[pl.ds(0, rpt)])
    pltpu.sync_copy(res_hbm.at[pl.ds(gbase, rpt)], ra1.at[pl.ds(0, rpt)])

    def _fin(j, carry):
        cntv = jnp.maximum(ra2[j], 1.0)
        xo = m_v[j] / cntv + ra0[j]
        xo = jnp.where(xo > 0, xo, jnp.exp(xo) - 1.0)
        xo = xo + ra1[j]
        xo = jnp.where(xo > 0, xo, jnp.exp(xo) - 1.0)
        m_v[j + rpt] = xo
        return carry
    lax.fori_loop(0, rpt, _fin, 0)
    pltpu.sync_copy(m_v.at[pl.ds(rpt, rpt)], out_hbm.at[pl.ds(gbase, rpt)])


@functools.cache
def _get_edges():
    mesh = plsc.VectorSubcoreMesh(core_axis_name="c", subcore_axis_name="s",
                                  num_cores=2, num_subcores=16)
    idx_t = pltpu.VMEM((EPT,), jnp.int32)
    row_t = pltpu.VMEM((EPT, C), jnp.float32)
    return pl.kernel(
        _edges_body,
        mesh=mesh,
        compiler_params=pltpu.CompilerParams(use_tc_tiling_on_sc=False),
        out_type=jax.ShapeDtypeStruct((NODES, C), jnp.float32),
        scratch_types=[
            idx_t, idx_t, idx_t, idx_t,
            idx_t, idx_t, idx_t, idx_t,
            idx_t,
            pltpu.VMEM((EPT * 64,), jnp.float32),
            row_t, row_t, row_t, row_t,
            row_t, row_t, row_t, row_t,
            row_t,
            pltpu.VMEM_SHARED((GPS * V, C), jnp.float32),
            pltpu.VMEM_SHARED((V, C), jnp.float32),
            pltpu.SemaphoreType.DMA,
            pltpu.SemaphoreType.DMA,
        ],
    )


def kernel(x, edge_index, edge_attr, Wspline, Wroot, b, Wres, bres):
    n, v, cc, t = x.shape
    xg = x.transpose(3, 0, 1, 2).reshape(NODES, C)

    wflat = Wspline.transpose(1, 0, 2).reshape(C, NK * C)
    wcat = jnp.concatenate([wflat, Wroot, Wres.T], axis=1)
    b2 = b.reshape(1, C)
    bres2 = bres.reshape(1, C)

    ea = edge_attr[:NEB]
    ea0 = ea[:, 0:1]
    ea1 = ea[:, 1:2]
    src = edge_index[0, :NEB].reshape(NEB, 1)
    dst = edge_index[1, :NEB]

    eexp = jnp.repeat(jnp.eye(4, dtype=jnp.float32), 16, axis=1)

    y, root, res, basisb, idx4 = _prep(xg, wcat, b2, bres2, ea0, ea1,
                                       src, eexp)
    yflat = y.reshape(NODES * NK, C)
    idxf = idx4.T.reshape(-1)
    basisf = basisb.reshape(-1)

    out_node = _get_edges()(yflat, idxf, basisf, dst, root, res)

    return out_node.reshape(t, n, v, C).transpose(1, 2, 3, 0)


# speedup vs baseline: 67.4030x; 1.2253x over previous
import functools

import jax
import jax.numpy as jnp
from jax import lax
from jax.experimental import pallas as pl
from jax.experimental.pallas import tpu as pltpu
from jax.experimental.pallas import tpu_sc as plsc

K = 5
V = 512
C = 16
NG = 16
NEB = 8192
NODES = NG * V
NK = K * K
EPT = NEB // 16
GPS = NG // 2


def _prep_body(xg_ref, wcat_ref, b2_ref, bres2_ref, ea0_ref, ea1_ref,
               src_ref, eexp_ref, y_ref, tail_ref, idx4_ref):
    xg = xg_ref[...]
    p = lax.dot_general(xg, wcat_ref[...], (((1,), (0,)), ((), ())),
                        preferred_element_type=jnp.float32)
    y_ref[...] = p[:, :NK * C]
    root = p[:, NK * C:NK * C + C] + b2_ref[...]
    r = p[:, NK * C + C:] + bres2_ref[...]
    res = jnp.where(r > 0, r, jnp.exp(r) - 1.0)

    pos0 = ea0_ref[...] * (K - 1.0)
    pos1 = ea1_ref[...] * (K - 1.0)
    i0f = jnp.clip(jnp.floor(pos0), 0.0, K - 2.0)
    i1f = jnp.clip(jnp.floor(pos1), 0.0, K - 2.0)
    f0 = pos0 - i0f
    f1 = pos1 - i1f
    g0 = 1.0 - f0
    g1 = 1.0 - f1
    basis4 = jnp.concatenate([g0 * g1, f0 * g1, g0 * f1, f0 * f1], axis=1)
    basisb = jnp.dot(basis4, eexp_ref[...],
                     preferred_element_type=jnp.float32)
    zer = jnp.zeros(basis4.shape[:1] + (2 * C,), jnp.float32)
    tail_ref[...] = jnp.concatenate([root, res, basisb, zer], axis=1)
    base = (src_ref[...] * NK + i0f.astype(jnp.int32)
            + i1f.astype(jnp.int32) * K)
    idx4_ref[...] = jnp.concatenate(
        [base, base + 1, base + K, base + K + 1], axis=1)


_PREP_BLK = 1024
_prep = pl.pallas_call(
    _prep_body,
    grid=(NODES // _PREP_BLK,),
    in_specs=[
        pl.BlockSpec((_PREP_BLK, C), lambda i: (i, 0)),
        pl.BlockSpec((C, 432), lambda i: (0, 0)),
        pl.BlockSpec((1, C), lambda i: (0, 0)),
        pl.BlockSpec((1, C), lambda i: (0, 0)),
        pl.BlockSpec((_PREP_BLK, 1), lambda i: (i, 0)),
        pl.BlockSpec((_PREP_BLK, 1), lambda i: (i, 0)),
        pl.BlockSpec((_PREP_BLK, 1), lambda i: (i, 0)),
        pl.BlockSpec((4, 64), lambda i: (0, 0)),
    ],
    out_specs=[
        pl.BlockSpec((_PREP_BLK, NK * C), lambda i: (i, 0)),
        pl.BlockSpec((_PREP_BLK, 128), lambda i: (i, 0)),
        pl.BlockSpec((_PREP_BLK, 4), lambda i: (i, 0)),
    ],
    out_shape=(
        jax.ShapeDtypeStruct((NODES, NK * C), jnp.float32),
        jax.ShapeDtypeStruct((NODES, 128), jnp.float32),
        jax.ShapeDtypeStruct((NEB, 4), jnp.int32),
    ),
)


def _edges_body(y_hbm, idxf_hbm, tail_hbm, dst_hbm,
                out_hbm,
                ia0, ia1, ia2, ia3, ib0, ib1, ib2, ib3, dst_v, basis_v,
                ra0, ra1, ra2, ra3, rb0, rb1, rb2, rb3,
                m_v, agg_sp, cnt_sp, sem_a, sem_b):
    c = lax.axis_index("c")
    s = lax.axis_index("s")
    e0 = s * EPT
    rpt = GPS * V // 16

    idx_a = (ia0, ia1, ia2, ia3)
    idx_b = (ib0, ib1, ib2, ib3)
    rows_a = (ra0, ra1, ra2, ra3)
    rows_b = (rb0, rb1, rb2, rb3)

    def _fill(j, carry):
        m_v[j] = jnp.zeros((C,), jnp.float32)
        ra0[j] = jnp.ones((C,), jnp.float32)
        ra0[j + rpt] = jnp.ones((C,), jnp.float32)
        return carry
    lax.fori_loop(0, rpt, _fill, 0)

    pltpu.sync_copy(m_v.at[pl.ds(0, rpt)], agg_sp.at[pl.ds(s * rpt, rpt)])
    pltpu.sync_copy(m_v.at[pl.ds(0, V // 16)],
                    cnt_sp.at[pl.ds(s * (V // 16), V // 16)])

    for corner in range(4):
        pltpu.sync_copy(idxf_hbm.at[pl.ds(corner * NEB + e0, EPT)],
                        idx_a[corner])
    pltpu.sync_copy(dst_hbm.at[pl.ds(e0, EPT)], dst_v)
    pltpu.sync_copy(tail_hbm.at[pl.ds(e0, EPT), pl.ds(2 * C, 4 * C)],
                    basis_v)

    def _addv(dref, sref, nchunks, val):
        def f(j, carry):
            dref[pl.ds(j * 16, 16)] = sref[pl.ds(j * 16, 16)] + val
            return carry
        lax.fori_loop(0, nchunks, f, 0)

    for corner in range(4):
        _addv(idx_a[corner], idx_a[corner], EPT // 16, c * (GPS * V * NK))
    plsc.subcore_barrier()

    pltpu.sync_copy(ra0, cnt_sp.at[dst_v], add=True)

    bufs = ((idx_a, rows_a, sem_a), (idx_b, rows_b, sem_b))
    descs = [pltpu.async_copy(y_hbm.at[ix], r, sem_a)
             for ix, r in zip(idx_a, rows_a)]
    for g in range(GPS):
        cur_i, cur_r, _ = bufs[g % 2]
        nxt_i, nxt_r, nxt_s = bufs[(g + 1) % 2]
        if g < GPS - 1:
            with jax.named_scope("idxprep"):
                for corner in range(4):
                    _addv(nxt_i[corner], cur_i[corner], EPT // 16, V * NK)
        with jax.named_scope("dwait"):
            for d in descs:
                d.wait()
        if g < GPS - 1:
            descs = [pltpu.async_copy(y_hbm.at[ix], r, nxt_s)
                     for ix, r in zip(nxt_i, nxt_r)]

        c0, c1, c2, c3 = cur_r

        with jax.named_scope("fma"):
            @plsc.parallel_loop(0, EPT, step=1, unroll=8)
            def _body(ee):
                m = (c0[ee] * basis_v[ee, pl.ds(0, 16)]
                     + c1[ee] * basis_v[ee, pl.ds(16, 16)]
                     + c2[ee] * basis_v[ee, pl.ds(32, 16)]
                     + c3[ee] * basis_v[ee, pl.ds(48, 16)])
                m_v[ee] = m

        with jax.named_scope("scat"):
            pltpu.sync_copy(m_v, agg_sp.at[dst_v], add=True)
        if g < GPS - 1:
            with jax.named_scope("dstinc"):
                _addv(dst_v, dst_v, EPT // 16, V)

    plsc.subcore_barrier()

    row0 = s * rpt
    gbase = c * (GPS * V) + row0
    v_off = lax.rem(s, 2) * rpt
    pltpu.sync_copy(agg_sp.at[pl.ds(row0, rpt)], m_v.at[pl.ds(0, rpt)])
    pltpu.sync_copy(cnt_sp.at[pl.ds(v_off, rpt)], ra2.at[pl.ds(0, rpt)])
    pltpu.sync_copy(tail_hbm.at[pl.ds(gbase, rpt), pl.ds(0, C)],
                    ra0.at[pl.ds(0, rpt)])
    pltpu.sync_copy(tail_hbm.at[pl.ds(gbase, rpt), pl.ds(C, C)],
                    ra1.at[pl.ds(0, rpt)])

    def _fin(j, carry):
        cntv = jnp.maximum(ra2[j], 1.0)
        xo = m_v[j] / cntv + ra0[j]
        xo = jnp.where(xo > 0, xo, jnp.exp(xo) - 1.0)
        xo = xo + ra1[j]
        xo = jnp.where(xo > 0, xo, jnp.exp(xo) - 1.0)
        m_v[j + rpt] = xo
        return carry
    lax.fori_loop(0, rpt, _fin, 0)
    pltpu.sync_copy(m_v.at[pl.ds(rpt, rpt)], out_hbm.at[pl.ds(gbase, rpt)])


@functools.cache
def _get_edges():
    mesh = plsc.VectorSubcoreMesh(core_axis_name="c", subcore_axis_name="s",
                                  num_cores=2, num_subcores=16)
    idx_t = pltpu.VMEM((EPT,), jnp.int32)
    row_t = pltpu.VMEM((EPT, C), jnp.float32)
    return pl.kernel(
        _edges_body,
        mesh=mesh,
        compiler_params=pltpu.CompilerParams(use_tc_tiling_on_sc=False),
        out_type=jax.ShapeDtypeStruct((NODES, C), jnp.float32),
        scratch_types=[
            idx_t, idx_t, idx_t, idx_t,
            idx_t, idx_t, idx_t, idx_t,
            idx_t,
            pltpu.VMEM((EPT, 64), jnp.float32),
            row_t, row_t, row_t, row_t,
            row_t, row_t, row_t, row_t,
            row_t,
            pltpu.VMEM_SHARED((GPS * V, C), jnp.float32),
            pltpu.VMEM_SHARED((V, C), jnp.float32),
            pltpu.SemaphoreType.DMA,
            pltpu.SemaphoreType.DMA,
        ],
    )


def kernel(x, edge_index, edge_attr, Wspline, Wroot, b, Wres, bres):
    n, v, cc, t = x.shape
    xg = x.transpose(3, 0, 1, 2).reshape(NODES, C)

    wflat = Wspline.transpose(1, 0, 2).reshape(C, NK * C)
    wcat = jnp.concatenate([wflat, Wroot, Wres.T], axis=1)
    b2 = b.reshape(1, C)
    bres2 = bres.reshape(1, C)

    ea = edge_attr[:NEB]
    ea0 = ea[:, 0:1]
    ea1 = ea[:, 1:2]
    src = edge_index[0, :NEB].reshape(NEB, 1)
    dst = edge_index[1, :NEB]

    eexp = jnp.repeat(jnp.eye(4, dtype=jnp.float32), 16, axis=1)

    y, tail, idx4 = _prep(xg, wcat, b2, bres2, ea0, ea1, src, eexp)
    yflat = y.reshape(NODES * NK, C)
    idxf = idx4.T.reshape(-1)

    out_node = _get_edges()(yflat, idxf, tail, dst)

    return out_node.reshape(t, n, v, C).transpose(1, 2, 3, 0)


# speedup vs baseline: 83.5510x; 1.2396x over previous
import functools

import jax
import jax.numpy as jnp
from jax import lax
from jax.experimental import pallas as pl
from jax.experimental.pallas import tpu as pltpu
from jax.experimental.pallas import tpu_sc as plsc

K = 5
V = 512
C = 16
NG = 16
NEB = 8192
NODES = NG * V
NK = K * K
EPT = NEB // 16
GPS = NG // 2


def _prep_body(xg_ref, wcat_ref, b2_ref, bres2_ref, y_ref, tail_ref):
    xg = xg_ref[...]
    p = lax.dot_general(xg, wcat_ref[...], (((1,), (0,)), ((), ())),
                        preferred_element_type=jnp.float32)
    y_ref[...] = p[:, :NK * C]
    root = p[:, NK * C:NK * C + C] + b2_ref[...]
    r = p[:, NK * C + C:] + bres2_ref[...]
    res = jnp.where(r > 0, r, jnp.exp(r) - 1.0)
    zer = jnp.zeros(root.shape[:1] + (6 * C,), jnp.float32)
    tail_ref[...] = jnp.concatenate([root, res, zer], axis=1)


_PREP_BLK = 1024
_prep = pl.pallas_call(
    _prep_body,
    grid=(NODES // _PREP_BLK,),
    in_specs=[
        pl.BlockSpec((_PREP_BLK, C), lambda i: (i, 0)),
        pl.BlockSpec((C, 432), lambda i: (0, 0)),
        pl.BlockSpec((1, C), lambda i: (0, 0)),
        pl.BlockSpec((1, C), lambda i: (0, 0)),
    ],
    out_specs=[
        pl.BlockSpec((_PREP_BLK, NK * C), lambda i: (i, 0)),
        pl.BlockSpec((_PREP_BLK, 128), lambda i: (i, 0)),
    ],
    out_shape=(
        jax.ShapeDtypeStruct((NODES, NK * C), jnp.float32),
        jax.ShapeDtypeStruct((NODES, 128), jnp.float32),
    ),
)


def _lane_bcast(vec, lane):
    return lax.gather(
        vec, jnp.full((16, 1), lane, jnp.int32),
        lax.GatherDimensionNumbers(offset_dims=(), collapsed_slice_dims=(0,),
                                   start_index_map=(0,)),
        (1,), mode=lax.GatherScatterMode.PROMISE_IN_BOUNDS)


def _edges_body(y_hbm, ea0_hbm, ea1_hbm, src_hbm, dst_hbm, tail_hbm,
                out_hbm,
                ia0, ia1, ia2, ia3, ib0, ib1, ib2, ib3, dst_v,
                bas0, bas1, bas2, bas3, ea0_v, ea1_v, src_v,
                ra0, ra1, ra2, ra3, rb0, rb1, rb2, rb3,
                m_v, agg_sp, cnt_sp, sem_a, sem_b):
    c = lax.axis_index("c")
    s = lax.axis_index("s")
    e0 = s * EPT
    rpt = GPS * V // 16

    idx_a = (ia0, ia1, ia2, ia3)
    idx_b = (ib0, ib1, ib2, ib3)
    rows_a = (ra0, ra1, ra2, ra3)
    rows_b = (rb0, rb1, rb2, rb3)

    @plsc.parallel_loop(0, rpt, step=1, unroll=4)
    def _fill(j):
        m_v[j] = jnp.zeros((C,), jnp.float32)
        ra0[j] = jnp.ones((C,), jnp.float32)
        ra0[j + rpt] = jnp.ones((C,), jnp.float32)

    pltpu.sync_copy(m_v.at[pl.ds(0, rpt)], agg_sp.at[pl.ds(s * rpt, rpt)])
    pltpu.sync_copy(m_v.at[pl.ds(0, V // 16)],
                    cnt_sp.at[pl.ds(s * (V // 16), V // 16)])

    pltpu.sync_copy(ea0_hbm.at[pl.ds(e0, EPT)], ea0_v)
    pltpu.sync_copy(ea1_hbm.at[pl.ds(e0, EPT)], ea1_v)
    pltpu.sync_copy(src_hbm.at[pl.ds(e0, EPT)], src_v)
    pltpu.sync_copy(dst_hbm.at[pl.ds(e0, EPT)], dst_v)

    off0 = c * (GPS * V * NK)

    @plsc.parallel_loop(0, EPT // 16, step=1, unroll=2)
    def _setup(j):
        sl = pl.ds(j * 16, 16)
        p0 = ea0_v[sl] * (K - 1.0)
        p1 = ea1_v[sl] * (K - 1.0)
        i0 = jnp.minimum(p0.astype(jnp.int32), K - 2)
        i1 = jnp.minimum(p1.astype(jnp.int32), K - 2)
        f0 = p0 - i0.astype(jnp.float32)
        f1 = p1 - i1.astype(jnp.float32)
        g0 = 1.0 - f0
        g1 = 1.0 - f1
        bas0[sl] = g0 * g1
        bas1[sl] = f0 * g1
        bas2[sl] = g0 * f1
        bas3[sl] = f0 * f1
        wib = src_v[sl] * NK + i0 + i1 * K + off0
        ia0[sl] = wib
        ia1[sl] = wib + 1
        ia2[sl] = wib + K
        ia3[sl] = wib + K + 1

    def _addv(dref, sref, nchunks, val):
        @plsc.parallel_loop(0, nchunks, step=1, unroll=2)
        def f(j):
            dref[pl.ds(j * 16, 16)] = sref[pl.ds(j * 16, 16)] + val

    plsc.subcore_barrier()

    pltpu.sync_copy(ra0, cnt_sp.at[dst_v], add=True)

    bufs = ((idx_a, rows_a, sem_a), (idx_b, rows_b, sem_b))
    descs = [pltpu.async_copy(y_hbm.at[ix], r, sem_a)
             for ix, r in zip(idx_a, rows_a)]
    for g in range(GPS):
        cur_i, cur_r, _ = bufs[g % 2]
        nxt_i, nxt_r, nxt_s = bufs[(g + 1) % 2]
        if g < GPS - 1:
            with jax.named_scope("idxprep"):
                for corner in range(4):
                    _addv(nxt_i[corner], cur_i[corner], EPT // 16, V * NK)
        with jax.named_scope("dwait"):
            for d in descs:
                d.wait()
        if g < GPS - 1:
            descs = [pltpu.async_copy(y_hbm.at[ix], r, nxt_s)
                     for ix, r in zip(nxt_i, nxt_r)]

        c0, c1, c2, c3 = cur_r

        with jax.named_scope("fma"):
            @plsc.parallel_loop(0, EPT // 16, step=1, unroll=1)
            def _body(ch):
                base = ch * 16
                sl = pl.ds(base, 16)
                b0 = bas0[sl]
                b1 = bas1[sl]
                b2v = bas2[sl]
                b3 = bas3[sl]
                for u in range(16):
                    ee = base + u
                    m = (c0[ee] * _lane_bcast(b0, u)
                         + c1[ee] * _lane_bcast(b1, u)
                         + c2[ee] * _lane_bcast(b2v, u)
                         + c3[ee] * _lane_bcast(b3, u))
                    m_v[ee] = m

        with jax.named_scope("scat"):
            pltpu.sync_copy(m_v, agg_sp.at[dst_v], add=True)
        if g < GPS - 1:
            with jax.named_scope("dstinc"):
                _addv(dst_v, dst_v, EPT // 16, V)

    plsc.subcore_barrier()

    row0 = s * rpt
    gbase = c * (GPS * V) + row0
    v_off = lax.rem(s, 2) * rpt
    pltpu.sync_copy(agg_sp.at[pl.ds(row0, rpt)], m_v.at[pl.ds(0, rpt)])
    pltpu.sync_copy(cnt_sp.at[pl.ds(v_off, rpt)], ra2.at[pl.ds(0, rpt)])
    pltpu.sync_copy(tail_hbm.at[pl.ds(gbase, rpt), pl.ds(0, C)],
                    ra0.at[pl.ds(0, rpt)])
    pltpu.sync_copy(tail_hbm.at[pl.ds(gbase, rpt), pl.ds(C, C)],
                    ra1.at[pl.ds(0, rpt)])

    def _fin(j, carry):
        cntv = jnp.maximum(ra2[j], 1.0)
        xo = m_v[j] / cntv + ra0[j]
        xo = jnp.where(xo > 0, xo, jnp.exp(xo) - 1.0)
        xo = xo + ra1[j]
        xo = jnp.where(xo > 0, xo, jnp.exp(xo) - 1.0)
        m_v[j + rpt] = xo
        return carry
    lax.fori_loop(0, rpt, _fin, 0)
    pltpu.sync_copy(m_v.at[pl.ds(rpt, rpt)], out_hbm.at[pl.ds(gbase, rpt)])


@functools.cache
def _get_edges():
    mesh = plsc.VectorSubcoreMesh(core_axis_name="c", subcore_axis_name="s",
                                  num_cores=2, num_subcores=16)
    idx_t = pltpu.VMEM((EPT,), jnp.int32)
    vec_t = pltpu.VMEM((EPT,), jnp.float32)
    row_t = pltpu.VMEM((EPT, C), jnp.float32)
    return pl.kernel(
        _edges_body,
        mesh=mesh,
        compiler_params=pltpu.CompilerParams(use_tc_tiling_on_sc=False),
        out_type=jax.ShapeDtypeStruct((NODES, C), jnp.float32),
        scratch_types=[
            idx_t, idx_t, idx_t, idx_t,
            idx_t, idx_t, idx_t, idx_t,
            idx_t,
            vec_t, vec_t, vec_t, vec_t,
            vec_t, vec_t,
            idx_t,
            row_t, row_t, row_t, row_t,
            row_t, row_t, row_t, row_t,
            row_t,
            pltpu.VMEM_SHARED((GPS * V, C), jnp.float32),
            pltpu.VMEM_SHARED((V, C), jnp.float32),
            pltpu.SemaphoreType.DMA,
            pltpu.SemaphoreType.DMA,
        ],
    )


def kernel(x, edge_index, edge_attr, Wspline, Wroot, b, Wres, bres):
    n, v, cc, t = x.shape
    xg = x.transpose(3, 0, 1, 2).reshape(NODES, C)

    wflat = Wspline.transpose(1, 0, 2).reshape(C, NK * C)
    wcat = jnp.concatenate([wflat, Wroot, Wres.T], axis=1)
    b2 = b.reshape(1, C)
    bres2 = bres.reshape(1, C)

    ea0 = edge_attr[:NEB, 0]
    ea1 = edge_attr[:NEB, 1]
    src = edge_index[0, :NEB]
    dst = edge_index[1, :NEB]

    y, tail = _prep(xg, wcat, b2, bres2)
    yflat = y.reshape(NODES * NK, C)

    out_node = _get_edges()(yflat, ea0, ea1, src, dst, tail)

    return out_node.reshape(t, n, v, C).transpose(1, 2, 3, 0)
